# baseline (speedup 1.0000x reference)
import functools

import jax
import jax.numpy as jnp
from jax import lax
from jax.experimental import pallas as pl
from jax.experimental.pallas import tpu as pltpu
from jax.experimental.pallas import tpu_sc as plsc

N = 10000
D = 256
H = 128
E = 160000
NC = 2
NS = 16
LANES = 16

E_TILE_DEG = E // (NC * NS)
E_TILE_AGG = E // NS
CHUNK = 50
G = 4
NGROUP = E_TILE_AGG // (G * CHUNK)
TB = 624
WB = 48
NWB = TB // WB
TAIL = N - NS * TB
TAIL_BASE = NS * TB

RB = 10
R = N // RB

_sc_mesh = plsc.VectorSubcoreMesh(core_axis_name="c", subcore_axis_name="s")
_sc_params = pltpu.CompilerParams(needs_layout_passes=False)


@functools.partial(
    pl.kernel,
    out_type=jax.ShapeDtypeStruct((NC * NS, 1, N), jnp.float32),
    mesh=_sc_mesh,
    compiler_params=_sc_params,
    scratch_types=[
        pltpu.VMEM((N,), jnp.float32),
        pltpu.VMEM((E_TILE_DEG,), jnp.int32),
    ],
)
def _deg_kernel(dst_hbm, out_hbm, hist, didx):
    c = lax.axis_index("c")
    s = lax.axis_index("s")
    wid = c * NS + s

    zeros16 = jnp.zeros((LANES,), jnp.float32)

    def zero_body(i, carry):
        hist[pl.ds(i * LANES, LANES)] = zeros16
        return carry

    lax.fori_loop(0, N // LANES, zero_body, 0)

    pltpu.sync_copy(dst_hbm.at[wid, 0], didx)

    ones16 = jnp.ones((LANES,), jnp.float32)
    n_full = E_TILE_DEG // LANES
    rem = E_TILE_DEG - n_full * LANES

    def body(i, carry):
        idx = didx[pl.ds(i * LANES, LANES)]
        plsc.addupdate_scatter(hist, [idx], ones16)
        return carry

    lax.fori_loop(0, n_full, body, 0)

    if rem:
        lane = lax.iota(jnp.int32, LANES)
        idx = didx[pl.ds(E_TILE_DEG - LANES, LANES)]
        vals = jnp.where(lane >= LANES - rem, 1.0, 0.0).astype(jnp.float32)
        idx = jnp.where(lane >= LANES - rem, idx, 0)
        plsc.addupdate_scatter(hist, [idx], vals)

    pltpu.sync_copy(hist, out_hbm.at[wid, 0])


@functools.partial(
    pl.kernel,
    out_type=(
        jax.ShapeDtypeStruct((N, H), jnp.float32),
        jax.ShapeDtypeStruct((N, H), jnp.float32),
    ),
    mesh=_sc_mesh,
    compiler_params=_sc_params,
    scratch_types=[
        pltpu.VMEM_SHARED((N, H), jnp.float32),
        pltpu.VMEM((G, 1, CHUNK), jnp.int32),
        pltpu.VMEM((G, 1, CHUNK), jnp.int32),
        [pltpu.VMEM((CHUNK, H), jnp.float32) for _ in range(G)],
        pltpu.VMEM((WB, H), jnp.float32),
        pltpu.SemaphoreType.DMA,
        pltpu.SemaphoreType.DMA,
    ],
)
def _agg_kernel(h0_hbm, h1_hbm, src_hbm, dst_hbm, out0_hbm, out1_hbm,
                acc, sidx, didx, rows, wbuf, gsem, ssem):
    c = lax.axis_index("c")
    s = lax.axis_index("s")

    zeros16 = jnp.zeros((LANES,), jnp.float32)

    def zero_row(i, carry):
        for k in range(H // LANES):
            wbuf[i, pl.ds(k * LANES, LANES)] = zeros16
        return carry

    lax.fori_loop(0, WB, zero_row, 0)

    base_row = s * TB
    for t in range(NWB):
        pltpu.sync_copy(wbuf, acc.at[pl.ds(base_row + t * WB, WB)])

    @pl.when(s == NS - 1)
    def _():
        pltpu.sync_copy(wbuf.at[pl.ds(0, TAIL)],
                        acc.at[pl.ds(TAIL_BASE, TAIL)])

    plsc.subcore_barrier()

    def group_body(g, carry):
        pltpu.sync_copy(src_hbm.at[s, g], sidx)
        pltpu.sync_copy(dst_hbm.at[s, g], didx)

        def run(h_hbm):
            gds = [
                pltpu.async_copy(h_hbm.at[sidx.at[k, 0]], rows[k], gsem)
                for k in range(G)
            ]
            sds = []
            for k in range(G):
                gds[k].wait()
                sds.append(
                    pltpu.async_copy(rows[k], acc.at[didx.at[k, 0]], ssem,
                                     add=True))
            for d in sds:
                d.wait()

        @pl.when(c == 0)
        def _():
            run(h0_hbm)

        @pl.when(c == 1)
        def _():
            run(h1_hbm)

        return carry

    lax.fori_loop(0, NGROUP, group_body, 0)
    plsc.subcore_barrier()

    def write_out(src_vmem, sl):
        @pl.when(c == 0)
        def _():
            pltpu.sync_copy(src_vmem, out0_hbm.at[sl])

        @pl.when(c == 1)
        def _():
            pltpu.sync_copy(src_vmem, out1_hbm.at[sl])

    for t in range(NWB):
        sl = pl.ds(base_row + t * WB, WB)
        pltpu.sync_copy(acc.at[sl], wbuf)
        write_out(wbuf, sl)

    @pl.when(s == NS - 1)
    def _():
        sl = pl.ds(TAIL_BASE, TAIL)
        pltpu.sync_copy(acc.at[sl], wbuf.at[pl.ds(0, TAIL)])
        write_out(wbuf.at[pl.ds(0, TAIL)], sl)


def _dinv_from_parts(deg_ref):
    deg = deg_ref[...].sum(axis=(0, 1, 2)) + 1.0
    return lax.rsqrt(deg)


def _tc1_body(x_ref, w_ref, deg_ref, h0_ref, h1_ref):
    dinv = _dinv_from_parts(deg_ref)[:, None]
    h = jnp.dot(x_ref[...], w_ref[...], preferred_element_type=jnp.float32)
    hp = h * dinv
    h0_ref[...] = hp[:, :H]
    h1_ref[...] = hp[:, H:]


_tc1 = pl.pallas_call(
    _tc1_body,
    grid=(RB,),
    in_specs=[
        pl.BlockSpec((R, D), lambda r: (r, 0)),
        pl.BlockSpec((D, D), lambda r: (0, 0)),
        pl.BlockSpec((NC * NS, 1, 1, R), lambda r: (0, r, 0, 0)),
    ],
    out_specs=(
        pl.BlockSpec((R, H), lambda r: (r, 0)),
        pl.BlockSpec((R, H), lambda r: (r, 0)),
    ),
    out_shape=(
        jax.ShapeDtypeStruct((N, H), jnp.float32),
        jax.ShapeDtypeStruct((N, H), jnp.float32),
    ),
)


def _tc2_body(a0_ref, a1_ref, p0_ref, p1_ref, deg_ref, b1_ref, w2_ref,
              o0_ref, o1_ref):
    dinv = _dinv_from_parts(deg_ref)[:, None]
    b1 = b1_ref[...]
    r0 = jnp.maximum(dinv * (a0_ref[...] + p0_ref[...]) + b1[0][None, :], 0.0)
    r1 = jnp.maximum(dinv * (a1_ref[...] + p1_ref[...]) + b1[1][None, :], 0.0)
    r = jnp.concatenate([r0, r1], axis=1)
    h = jnp.dot(r, w2_ref[...], preferred_element_type=jnp.float32)
    hp = h * dinv
    o0_ref[...] = hp[:, :H]
    o1_ref[...] = hp[:, H:]


_tc2 = pl.pallas_call(
    _tc2_body,
    grid=(RB,),
    in_specs=[
        pl.BlockSpec((R, H), lambda r: (r, 0)),
        pl.BlockSpec((R, H), lambda r: (r, 0)),
        pl.BlockSpec((R, H), lambda r: (r, 0)),
        pl.BlockSpec((R, H), lambda r: (r, 0)),
        pl.BlockSpec((NC * NS, 1, 1, R), lambda r: (0, r, 0, 0)),
        pl.BlockSpec((2, H), lambda r: (0, 0)),
        pl.BlockSpec((D, D), lambda r: (0, 0)),
    ],
    out_specs=(
        pl.BlockSpec((R, H), lambda r: (r, 0)),
        pl.BlockSpec((R, H), lambda r: (r, 0)),
    ),
    out_shape=(
        jax.ShapeDtypeStruct((N, H), jnp.float32),
        jax.ShapeDtypeStruct((N, H), jnp.float32),
    ),
)


def _tc3_body(a0_ref, a1_ref, p0_ref, p1_ref, deg_ref, b2_ref, out_ref):
    dinv = _dinv_from_parts(deg_ref)[:, None]
    b2 = b2_ref[...]
    o0 = dinv * (a0_ref[...] + p0_ref[...]) + b2[0][None, :]
    o1 = dinv * (a1_ref[...] + p1_ref[...]) + b2[1][None, :]
    out_ref[...] = jnp.concatenate([o0, o1], axis=1)


_tc3 = pl.pallas_call(
    _tc3_body,
    grid=(RB,),
    in_specs=[
        pl.BlockSpec((R, H), lambda r: (r, 0)),
        pl.BlockSpec((R, H), lambda r: (r, 0)),
        pl.BlockSpec((R, H), lambda r: (r, 0)),
        pl.BlockSpec((R, H), lambda r: (r, 0)),
        pl.BlockSpec((NC * NS, 1, 1, R), lambda r: (0, r, 0, 0)),
        pl.BlockSpec((2, H), lambda r: (0, 0)),
    ],
    out_specs=pl.BlockSpec((R, D), lambda r: (r, 0)),
    out_shape=jax.ShapeDtypeStruct((N, D), jnp.float32),
)


def kernel(x, edge_index, W1, b1, W2, b2):
    src = edge_index[0].astype(jnp.int32)
    dst = edge_index[1].astype(jnp.int32)

    deg_parts = _deg_kernel(dst.reshape(NC * NS, 1, E_TILE_DEG))
    deg_parts = deg_parts.reshape(NC * NS, RB, 1, R)

    src_r = src.reshape(NS, NGROUP, G, 1, CHUNK)
    dst_r = dst.reshape(NS, NGROUP, G, 1, CHUNK)

    h0, h1 = _tc1(x, W1, deg_parts)
    a0, a1 = _agg_kernel(h0, h1, src_r, dst_r)
    g0, g1 = _tc2(a0, a1, h0, h1, deg_parts, b1.reshape(2, H), W2)
    c0, c1 = _agg_kernel(g0, g1, src_r, dst_r)
    return _tc3(c0, c1, g0, g1, deg_parts, b2.reshape(2, H))


# speedup vs baseline: 1.0599x; 1.0599x over previous
import functools

import jax
import jax.numpy as jnp
from jax import lax
from jax.experimental import pallas as pl
from jax.experimental.pallas import tpu as pltpu
from jax.experimental.pallas import tpu_sc as plsc

N = 10000
D = 256
H = 128
E = 160000
NC = 2
NS = 16
LANES = 16

E_TILE_DEG = E // (NC * NS)
E_TILE_AGG = E // NS
CHUNK = 80
G = 3
NCHUNK = E_TILE_AGG // CHUNK
NGROUP = NCHUNK // G
GTAIL = NCHUNK - NGROUP * G
TB = 624
WB = 48
NWB = TB // WB
TAIL = N - NS * TB
TAIL_BASE = NS * TB

RB = 10
R = N // RB

_sc_mesh = plsc.VectorSubcoreMesh(core_axis_name="c", subcore_axis_name="s")
_sc_params = pltpu.CompilerParams(needs_layout_passes=False)


@functools.partial(
    pl.kernel,
    out_type=jax.ShapeDtypeStruct((NC * NS, 1, N), jnp.float32),
    mesh=_sc_mesh,
    compiler_params=_sc_params,
    scratch_types=[
        pltpu.VMEM((N,), jnp.float32),
        pltpu.VMEM((E_TILE_DEG,), jnp.int32),
    ],
)
def _deg_kernel(dst_hbm, out_hbm, hist, didx):
    c = lax.axis_index("c")
    s = lax.axis_index("s")
    wid = c * NS + s

    zeros16 = jnp.zeros((LANES,), jnp.float32)

    def zero_body(i, carry):
        hist[pl.ds(i * LANES, LANES)] = zeros16
        return carry

    lax.fori_loop(0, N // LANES, zero_body, 0)

    pltpu.sync_copy(dst_hbm.at[wid, 0], didx)

    ones16 = jnp.ones((LANES,), jnp.float32)
    n_full = E_TILE_DEG // LANES
    rem = E_TILE_DEG - n_full * LANES

    def body(i, carry):
        idx = didx[pl.ds(i * LANES, LANES)]
        plsc.addupdate_scatter(hist, [idx], ones16)
        return carry

    lax.fori_loop(0, n_full, body, 0)

    if rem:
        lane = lax.iota(jnp.int32, LANES)
        idx = didx[pl.ds(E_TILE_DEG - LANES, LANES)]
        vals = jnp.where(lane >= LANES - rem, 1.0, 0.0).astype(jnp.float32)
        idx = jnp.where(lane >= LANES - rem, idx, 0)
        plsc.addupdate_scatter(hist, [idx], vals)

    pltpu.sync_copy(hist, out_hbm.at[wid, 0])


@functools.partial(
    pl.kernel,
    out_type=(
        jax.ShapeDtypeStruct((N, H), jnp.float32),
        jax.ShapeDtypeStruct((N, H), jnp.float32),
    ),
    mesh=_sc_mesh,
    compiler_params=_sc_params,
    scratch_types=[
        pltpu.VMEM_SHARED((N, H), jnp.float32),
        pltpu.VMEM((G, 1, CHUNK), jnp.int32),
        pltpu.VMEM((G, 1, CHUNK), jnp.int32),
        [pltpu.VMEM((CHUNK, H), jnp.float32) for _ in range(G)],
        pltpu.VMEM((WB, H), jnp.float32),
        pltpu.SemaphoreType.DMA,
        pltpu.SemaphoreType.DMA,
    ],
)
def _agg_kernel(h0_hbm, h1_hbm, src_hbm, dst_hbm, out0_hbm, out1_hbm,
                acc, sidx, didx, rows, wbuf, gsem, ssem):
    c = lax.axis_index("c")
    s = lax.axis_index("s")

    zeros16 = jnp.zeros((LANES,), jnp.float32)

    def zero_row(i, carry):
        for k in range(H // LANES):
            wbuf[i, pl.ds(k * LANES, LANES)] = zeros16
        return carry

    lax.fori_loop(0, WB, zero_row, 0)

    base_row = s * TB
    for t in range(NWB):
        pltpu.sync_copy(wbuf, acc.at[pl.ds(base_row + t * WB, WB)])

    @pl.when(s == NS - 1)
    def _():
        pltpu.sync_copy(wbuf.at[pl.ds(0, TAIL)],
                        acc.at[pl.ds(TAIL_BASE, TAIL)])

    plsc.subcore_barrier()

    def run(h_hbm, nk):
        gds = [
            pltpu.async_copy(h_hbm.at[sidx.at[k, 0]], rows[k], gsem)
            for k in range(nk)
        ]
        sds = []
        for k in range(nk):
            gds[k].wait()
            sds.append(
                pltpu.async_copy(rows[k], acc.at[didx.at[k, 0]], ssem,
                                 add=True))
        for d in sds:
            d.wait()

    def group_body(g, carry):
        pltpu.sync_copy(src_hbm.at[s, pl.ds(g * G, G)], sidx)
        pltpu.sync_copy(dst_hbm.at[s, pl.ds(g * G, G)], didx)

        @pl.when(c == 0)
        def _():
            run(h0_hbm, G)

        @pl.when(c == 1)
        def _():
            run(h1_hbm, G)

        return carry

    lax.fori_loop(0, NGROUP, group_body, 0)

    if GTAIL:
        pltpu.sync_copy(src_hbm.at[s, pl.ds(NGROUP * G, GTAIL)],
                        sidx.at[pl.ds(0, GTAIL)])
        pltpu.sync_copy(dst_hbm.at[s, pl.ds(NGROUP * G, GTAIL)],
                        didx.at[pl.ds(0, GTAIL)])

        @pl.when(c == 0)
        def _():
            run(h0_hbm, GTAIL)

        @pl.when(c == 1)
        def _():
            run(h1_hbm, GTAIL)

    plsc.subcore_barrier()

    def write_out(src_vmem, sl):
        @pl.when(c == 0)
        def _():
            pltpu.sync_copy(src_vmem, out0_hbm.at[sl])

        @pl.when(c == 1)
        def _():
            pltpu.sync_copy(src_vmem, out1_hbm.at[sl])

    for t in range(NWB):
        sl = pl.ds(base_row + t * WB, WB)
        pltpu.sync_copy(acc.at[sl], wbuf)
        write_out(wbuf, sl)

    @pl.when(s == NS - 1)
    def _():
        sl = pl.ds(TAIL_BASE, TAIL)
        pltpu.sync_copy(acc.at[sl], wbuf.at[pl.ds(0, TAIL)])
        write_out(wbuf.at[pl.ds(0, TAIL)], sl)


def _dinv_from_parts(deg_ref):
    deg = deg_ref[...].sum(axis=(0, 1, 2)) + 1.0
    return lax.rsqrt(deg)


def _tc1_body(x_ref, w_ref, deg_ref, h0_ref, h1_ref):
    dinv = _dinv_from_parts(deg_ref)[:, None]
    h = jnp.dot(x_ref[...], w_ref[...], preferred_element_type=jnp.float32)
    hp = h * dinv
    h0_ref[...] = hp[:, :H]
    h1_ref[...] = hp[:, H:]


_tc1 = pl.pallas_call(
    _tc1_body,
    grid=(RB,),
    in_specs=[
        pl.BlockSpec((R, D), lambda r: (r, 0)),
        pl.BlockSpec((D, D), lambda r: (0, 0)),
        pl.BlockSpec((NC * NS, 1, 1, R), lambda r: (0, r, 0, 0)),
    ],
    out_specs=(
        pl.BlockSpec((R, H), lambda r: (r, 0)),
        pl.BlockSpec((R, H), lambda r: (r, 0)),
    ),
    out_shape=(
        jax.ShapeDtypeStruct((N, H), jnp.float32),
        jax.ShapeDtypeStruct((N, H), jnp.float32),
    ),
)


def _tc2_body(a0_ref, a1_ref, p0_ref, p1_ref, deg_ref, b1_ref, w2_ref,
              o0_ref, o1_ref):
    dinv = _dinv_from_parts(deg_ref)[:, None]
    b1 = b1_ref[...]
    r0 = jnp.maximum(dinv * (a0_ref[...] + p0_ref[...]) + b1[0][None, :], 0.0)
    r1 = jnp.maximum(dinv * (a1_ref[...] + p1_ref[...]) + b1[1][None, :], 0.0)
    r = jnp.concatenate([r0, r1], axis=1)
    h = jnp.dot(r, w2_ref[...], preferred_element_type=jnp.float32)
    hp = h * dinv
    o0_ref[...] = hp[:, :H]
    o1_ref[...] = hp[:, H:]


_tc2 = pl.pallas_call(
    _tc2_body,
    grid=(RB,),
    in_specs=[
        pl.BlockSpec((R, H), lambda r: (r, 0)),
        pl.BlockSpec((R, H), lambda r: (r, 0)),
        pl.BlockSpec((R, H), lambda r: (r, 0)),
        pl.BlockSpec((R, H), lambda r: (r, 0)),
        pl.BlockSpec((NC * NS, 1, 1, R), lambda r: (0, r, 0, 0)),
        pl.BlockSpec((2, H), lambda r: (0, 0)),
        pl.BlockSpec((D, D), lambda r: (0, 0)),
    ],
    out_specs=(
        pl.BlockSpec((R, H), lambda r: (r, 0)),
        pl.BlockSpec((R, H), lambda r: (r, 0)),
    ),
    out_shape=(
        jax.ShapeDtypeStruct((N, H), jnp.float32),
        jax.ShapeDtypeStruct((N, H), jnp.float32),
    ),
)


def _tc3_body(a0_ref, a1_ref, p0_ref, p1_ref, deg_ref, b2_ref, out_ref):
    dinv = _dinv_from_parts(deg_ref)[:, None]
    b2 = b2_ref[...]
    o0 = dinv * (a0_ref[...] + p0_ref[...]) + b2[0][None, :]
    o1 = dinv * (a1_ref[...] + p1_ref[...]) + b2[1][None, :]
    out_ref[...] = jnp.concatenate([o0, o1], axis=1)


_tc3 = pl.pallas_call(
    _tc3_body,
    grid=(RB,),
    in_specs=[
        pl.BlockSpec((R, H), lambda r: (r, 0)),
        pl.BlockSpec((R, H), lambda r: (r, 0)),
        pl.BlockSpec((R, H), lambda r: (r, 0)),
        pl.BlockSpec((R, H), lambda r: (r, 0)),
        pl.BlockSpec((NC * NS, 1, 1, R), lambda r: (0, r, 0, 0)),
        pl.BlockSpec((2, H), lambda r: (0, 0)),
    ],
    out_specs=pl.BlockSpec((R, D), lambda r: (r, 0)),
    out_shape=jax.ShapeDtypeStruct((N, D), jnp.float32),
)


def kernel(x, edge_index, W1, b1, W2, b2):
    src = edge_index[0].astype(jnp.int32)
    dst = edge_index[1].astype(jnp.int32)

    deg_parts = _deg_kernel(dst.reshape(NC * NS, 1, E_TILE_DEG))
    deg_parts = deg_parts.reshape(NC * NS, RB, 1, R)

    src_r = src.reshape(NS, NCHUNK, 1, CHUNK)
    dst_r = dst.reshape(NS, NCHUNK, 1, CHUNK)

    h0, h1 = _tc1(x, W1, deg_parts)
    a0, a1 = _agg_kernel(h0, h1, src_r, dst_r)
    g0, g1 = _tc2(a0, a1, h0, h1, deg_parts, b1.reshape(2, H), W2)
    c0, c1 = _agg_kernel(g0, g1, src_r, dst_r)
    return _tc3(c0, c1, g0, g1, deg_parts, b2.reshape(2, H))


# speedup vs baseline: 1.0676x; 1.0072x over previous
import functools

import jax
import jax.numpy as jnp
from jax import lax
from jax.experimental import pallas as pl
from jax.experimental.pallas import tpu as pltpu
from jax.experimental.pallas import tpu_sc as plsc

N = 10000
D = 256
H = 128
E = 160000
NC = 2
NS = 16
LANES = 16

E_TILE_DEG = E // (NC * NS)
E_TILE_AGG = E // NS
CHUNK = 80
G = 3
NCHUNK = E_TILE_AGG // CHUNK
NGROUP = NCHUNK // G
GTAIL = NCHUNK - NGROUP * G
TB = 624
WB = 24
NWB = TB // WB
TAIL = N - NS * TB
TAIL_BASE = NS * TB

RB = 10
R = N // RB

_sc_mesh = plsc.VectorSubcoreMesh(core_axis_name="c", subcore_axis_name="s")
_sc_params = pltpu.CompilerParams(needs_layout_passes=False)


@functools.partial(
    pl.kernel,
    out_type=jax.ShapeDtypeStruct((NC * NS, 1, N), jnp.float32),
    mesh=_sc_mesh,
    compiler_params=_sc_params,
    scratch_types=[
        pltpu.VMEM((N,), jnp.float32),
        pltpu.VMEM((E_TILE_DEG,), jnp.int32),
    ],
)
def _deg_kernel(dst_hbm, out_hbm, hist, didx):
    c = lax.axis_index("c")
    s = lax.axis_index("s")
    wid = c * NS + s

    zeros16 = jnp.zeros((LANES,), jnp.float32)

    def zero_body(i, carry):
        hist[pl.ds(i * LANES, LANES)] = zeros16
        return carry

    lax.fori_loop(0, N // LANES, zero_body, 0)

    pltpu.sync_copy(dst_hbm.at[wid, 0], didx)

    ones16 = jnp.ones((LANES,), jnp.float32)
    n_full = E_TILE_DEG // LANES
    rem = E_TILE_DEG - n_full * LANES

    def body(i, carry):
        idx = didx[pl.ds(i * LANES, LANES)]
        plsc.addupdate_scatter(hist, [idx], ones16)
        return carry

    lax.fori_loop(0, n_full, body, 0)

    if rem:
        lane = lax.iota(jnp.int32, LANES)
        idx = didx[pl.ds(E_TILE_DEG - LANES, LANES)]
        vals = jnp.where(lane >= LANES - rem, 1.0, 0.0).astype(jnp.float32)
        idx = jnp.where(lane >= LANES - rem, idx, 0)
        plsc.addupdate_scatter(hist, [idx], vals)

    pltpu.sync_copy(hist, out_hbm.at[wid, 0])


@functools.partial(
    pl.kernel,
    out_type=(
        jax.ShapeDtypeStruct((N, H), jnp.float32),
        jax.ShapeDtypeStruct((N, H), jnp.float32),
    ),
    mesh=_sc_mesh,
    compiler_params=_sc_params,
    scratch_types=[
        pltpu.VMEM_SHARED((N, H), jnp.float32),
        pltpu.VMEM((G, 1, CHUNK), jnp.int32),
        pltpu.VMEM((G, 1, CHUNK), jnp.int32),
        [pltpu.VMEM((CHUNK, H), jnp.float32) for _ in range(G)],
        pltpu.VMEM((WB, H), jnp.float32),
        pltpu.VMEM((WB, H), jnp.float32),
        pltpu.SemaphoreType.DMA,
        pltpu.SemaphoreType.DMA,
    ],
)
def _agg_kernel(h0_hbm, h1_hbm, src_hbm, dst_hbm, out0_hbm, out1_hbm,
                acc, sidx, didx, rows, wbuf_a, wbuf_b, gsem, ssem):
    c = lax.axis_index("c")
    s = lax.axis_index("s")

    zeros16 = jnp.zeros((LANES,), jnp.float32)

    def zero_row(i, carry):
        for k in range(H // LANES):
            wbuf_a[i, pl.ds(k * LANES, LANES)] = zeros16
        return carry

    lax.fori_loop(0, WB, zero_row, 0)

    base_row = s * TB
    zds = []
    for t in range(NWB):
        if t >= 2:
            zds[t - 2].wait()
        zds.append(pltpu.async_copy(
            wbuf_a, acc.at[pl.ds(base_row + t * WB, WB)], gsem))
    zds[NWB - 2].wait()
    zds[NWB - 1].wait()

    @pl.when(s == NS - 1)
    def _():
        pltpu.sync_copy(wbuf_a.at[pl.ds(0, TAIL)],
                        acc.at[pl.ds(TAIL_BASE, TAIL)])

    plsc.subcore_barrier()

    def run(h_hbm, nk):
        gds = [
            pltpu.async_copy(h_hbm.at[sidx.at[k, 0]], rows[k], gsem)
            for k in range(nk)
        ]
        sds = []
        for k in range(nk):
            gds[k].wait()
            sds.append(
                pltpu.async_copy(rows[k], acc.at[didx.at[k, 0]], ssem,
                                 add=True))
        for d in sds:
            d.wait()

    def group_body(g, carry):
        pltpu.sync_copy(src_hbm.at[s, pl.ds(g * G, G)], sidx)
        pltpu.sync_copy(dst_hbm.at[s, pl.ds(g * G, G)], didx)

        @pl.when(c == 0)
        def _():
            run(h0_hbm, G)

        @pl.when(c == 1)
        def _():
            run(h1_hbm, G)

        return carry

    lax.fori_loop(0, NGROUP, group_body, 0)

    if GTAIL:
        pltpu.sync_copy(src_hbm.at[s, pl.ds(NGROUP * G, GTAIL)],
                        sidx.at[pl.ds(0, GTAIL)])
        pltpu.sync_copy(dst_hbm.at[s, pl.ds(NGROUP * G, GTAIL)],
                        didx.at[pl.ds(0, GTAIL)])

        @pl.when(c == 0)
        def _():
            run(h0_hbm, GTAIL)

        @pl.when(c == 1)
        def _():
            run(h1_hbm, GTAIL)

    plsc.subcore_barrier()

    def run_wb(out_hbm):
        wrs = []
        for t in range(NWB):
            sl = pl.ds(base_row + t * WB, WB)
            buf = wbuf_a if t % 2 == 0 else wbuf_b
            if t >= 2:
                wrs[t - 2].wait()
            pltpu.sync_copy(acc.at[sl], buf)
            wrs.append(pltpu.async_copy(buf, out_hbm.at[sl], ssem))
        wrs[NWB - 2].wait()
        wrs[NWB - 1].wait()

        @pl.when(s == NS - 1)
        def _():
            sl = pl.ds(TAIL_BASE, TAIL)
            pltpu.sync_copy(acc.at[sl], wbuf_a.at[pl.ds(0, TAIL)])
            pltpu.sync_copy(wbuf_a.at[pl.ds(0, TAIL)], out_hbm.at[sl])

    @pl.when(c == 0)
    def _():
        run_wb(out0_hbm)

    @pl.when(c == 1)
    def _():
        run_wb(out1_hbm)


def _dinv_from_parts(deg_ref):
    deg = deg_ref[...].sum(axis=(0, 1, 2)) + 1.0
    return lax.rsqrt(deg)


def _tc1_body(x_ref, w_ref, deg_ref, h0_ref, h1_ref):
    dinv = _dinv_from_parts(deg_ref)[:, None]
    h = jnp.dot(x_ref[...], w_ref[...], preferred_element_type=jnp.float32)
    hp = h * dinv
    h0_ref[...] = hp[:, :H]
    h1_ref[...] = hp[:, H:]


_tc1 = pl.pallas_call(
    _tc1_body,
    grid=(RB,),
    in_specs=[
        pl.BlockSpec((R, D), lambda r: (r, 0)),
        pl.BlockSpec((D, D), lambda r: (0, 0)),
        pl.BlockSpec((NC * NS, 1, 1, R), lambda r: (0, r, 0, 0)),
    ],
    out_specs=(
        pl.BlockSpec((R, H), lambda r: (r, 0)),
        pl.BlockSpec((R, H), lambda r: (r, 0)),
    ),
    out_shape=(
        jax.ShapeDtypeStruct((N, H), jnp.float32),
        jax.ShapeDtypeStruct((N, H), jnp.float32),
    ),
)


def _tc2_body(a0_ref, a1_ref, p0_ref, p1_ref, deg_ref, b1_ref, w2_ref,
              o0_ref, o1_ref):
    dinv = _dinv_from_parts(deg_ref)[:, None]
    b1 = b1_ref[...]
    r0 = jnp.maximum(dinv * (a0_ref[...] + p0_ref[...]) + b1[0][None, :], 0.0)
    r1 = jnp.maximum(dinv * (a1_ref[...] + p1_ref[...]) + b1[1][None, :], 0.0)
    r = jnp.concatenate([r0, r1], axis=1)
    h = jnp.dot(r, w2_ref[...], preferred_element_type=jnp.float32)
    hp = h * dinv
    o0_ref[...] = hp[:, :H]
    o1_ref[...] = hp[:, H:]


_tc2 = pl.pallas_call(
    _tc2_body,
    grid=(RB,),
    in_specs=[
        pl.BlockSpec((R, H), lambda r: (r, 0)),
        pl.BlockSpec((R, H), lambda r: (r, 0)),
        pl.BlockSpec((R, H), lambda r: (r, 0)),
        pl.BlockSpec((R, H), lambda r: (r, 0)),
        pl.BlockSpec((NC * NS, 1, 1, R), lambda r: (0, r, 0, 0)),
        pl.BlockSpec((2, H), lambda r: (0, 0)),
        pl.BlockSpec((D, D), lambda r: (0, 0)),
    ],
    out_specs=(
        pl.BlockSpec((R, H), lambda r: (r, 0)),
        pl.BlockSpec((R, H), lambda r: (r, 0)),
    ),
    out_shape=(
        jax.ShapeDtypeStruct((N, H), jnp.float32),
        jax.ShapeDtypeStruct((N, H), jnp.float32),
    ),
)


def _tc3_body(a0_ref, a1_ref, p0_ref, p1_ref, deg_ref, b2_ref, out_ref):
    dinv = _dinv_from_parts(deg_ref)[:, None]
    b2 = b2_ref[...]
    o0 = dinv * (a0_ref[...] + p0_ref[...]) + b2[0][None, :]
    o1 = dinv * (a1_ref[...] + p1_ref[...]) + b2[1][None, :]
    out_ref[...] = jnp.concatenate([o0, o1], axis=1)


_tc3 = pl.pallas_call(
    _tc3_body,
    grid=(RB,),
    in_specs=[
        pl.BlockSpec((R, H), lambda r: (r, 0)),
        pl.BlockSpec((R, H), lambda r: (r, 0)),
        pl.BlockSpec((R, H), lambda r: (r, 0)),
        pl.BlockSpec((R, H), lambda r: (r, 0)),
        pl.BlockSpec((NC * NS, 1, 1, R), lambda r: (0, r, 0, 0)),
        pl.BlockSpec((2, H), lambda r: (0, 0)),
    ],
    out_specs=pl.BlockSpec((R, D), lambda r: (r, 0)),
    out_shape=jax.ShapeDtypeStruct((N, D), jnp.float32),
)


def kernel(x, edge_index, W1, b1, W2, b2):
    src = edge_index[0].astype(jnp.int32)
    dst = edge_index[1].astype(jnp.int32)

    deg_parts = _deg_kernel(dst.reshape(NC * NS, 1, E_TILE_DEG))
    deg_parts = deg_parts.reshape(NC * NS, RB, 1, R)

    src_r = src.reshape(NS, NCHUNK, 1, CHUNK)
    dst_r = dst.reshape(NS, NCHUNK, 1, CHUNK)

    h0, h1 = _tc1(x, W1, deg_parts)
    a0, a1 = _agg_kernel(h0, h1, src_r, dst_r)
    g0, g1 = _tc2(a0, a1, h0, h1, deg_parts, b1.reshape(2, H), W2)
    c0, c1 = _agg_kernel(g0, g1, src_r, dst_r)
    return _tc3(c0, c1, g0, g1, deg_parts, b2.reshape(2, H))


# speedup vs baseline: 1.1773x; 1.1028x over previous
import functools

import jax
import jax.numpy as jnp
from jax import lax
from jax.experimental import pallas as pl
from jax.experimental.pallas import tpu as pltpu
from jax.experimental.pallas import tpu_sc as plsc

N = 10000
D = 256
H = 128
E = 160000
NC = 2
NS = 16
LANES = 16

E_TILE_DEG = E // (NC * NS)
E_TILE_AGG = E // NS
CHUNK = 80
G = 3
NCHUNK = E_TILE_AGG // CHUNK
NGROUP = NCHUNK // G
GTAIL = NCHUNK - NGROUP * G
TB = 624
WB = 24
NWB = TB // WB
TAIL = N - NS * TB
TAIL_BASE = NS * TB

RB = 10
R = N // RB

_sc_mesh = plsc.VectorSubcoreMesh(core_axis_name="c", subcore_axis_name="s")
_sc_params = pltpu.CompilerParams(needs_layout_passes=False)


@functools.partial(
    pl.kernel,
    out_type=jax.ShapeDtypeStruct((NC * NS, 1, N), jnp.float32),
    mesh=_sc_mesh,
    compiler_params=_sc_params,
    scratch_types=[
        pltpu.VMEM((N,), jnp.float32),
        pltpu.VMEM((E_TILE_DEG,), jnp.int32),
    ],
)
def _deg_kernel(dst_hbm, out_hbm, hist, didx):
    c = lax.axis_index("c")
    s = lax.axis_index("s")
    wid = c * NS + s

    zeros16 = jnp.zeros((LANES,), jnp.float32)

    def zero_body(i, carry):
        hist[pl.ds(i * LANES, LANES)] = zeros16
        return carry

    lax.fori_loop(0, N // LANES, zero_body, 0)

    pltpu.sync_copy(dst_hbm.at[wid, 0], didx)

    ones16 = jnp.ones((LANES,), jnp.float32)
    n_full = E_TILE_DEG // LANES
    rem = E_TILE_DEG - n_full * LANES

    def body(i, carry):
        idx = didx[pl.ds(i * LANES, LANES)]
        plsc.addupdate_scatter(hist, [idx], ones16)
        return carry

    lax.fori_loop(0, n_full, body, 0)

    if rem:
        lane = lax.iota(jnp.int32, LANES)
        idx = didx[pl.ds(E_TILE_DEG - LANES, LANES)]
        vals = jnp.where(lane >= LANES - rem, 1.0, 0.0).astype(jnp.float32)
        idx = jnp.where(lane >= LANES - rem, idx, 0)
        plsc.addupdate_scatter(hist, [idx], vals)

    pltpu.sync_copy(hist, out_hbm.at[wid, 0])


@functools.partial(
    pl.kernel,
    out_type=(
        jax.ShapeDtypeStruct((N, H), jnp.float32),
        jax.ShapeDtypeStruct((N, H), jnp.float32),
    ),
    mesh=_sc_mesh,
    compiler_params=_sc_params,
    scratch_types=[
        pltpu.VMEM_SHARED((N, H), jnp.float32),
        pltpu.VMEM((G, 1, CHUNK), jnp.int32),
        pltpu.VMEM((G, 1, CHUNK), jnp.int32),
        [pltpu.VMEM((CHUNK, H), jnp.float32) for _ in range(G)],
        pltpu.VMEM((WB, H), jnp.float32),
        pltpu.VMEM((WB, H), jnp.float32),
        pltpu.SemaphoreType.DMA,
        pltpu.SemaphoreType.DMA,
    ],
)
def _agg_kernel(h0_hbm, h1_hbm, src_hbm, dst_hbm, out0_hbm, out1_hbm,
                acc, sidx, didx, rows, wbuf_a, wbuf_b, gsem, ssem):
    c = lax.axis_index("c")
    s = lax.axis_index("s")

    zeros16 = jnp.zeros((LANES,), jnp.float32)

    def zero_row(i, carry):
        for k in range(H // LANES):
            wbuf_a[i, pl.ds(k * LANES, LANES)] = zeros16
        return carry

    lax.fori_loop(0, WB, zero_row, 0)

    base_row = s * TB
    zds = []
    for t in range(NWB):
        if t >= 2:
            zds[t - 2].wait()
        zds.append(pltpu.async_copy(
            wbuf_a, acc.at[pl.ds(base_row + t * WB, WB)], gsem))
    zds[NWB - 2].wait()
    zds[NWB - 1].wait()

    @pl.when(s == NS - 1)
    def _():
        pltpu.sync_copy(wbuf_a.at[pl.ds(0, TAIL)],
                        acc.at[pl.ds(TAIL_BASE, TAIL)])

    plsc.subcore_barrier()

    def run(h_hbm, nk):
        gds = [
            pltpu.async_copy(h_hbm.at[sidx.at[k, 0]], rows[k], gsem)
            for k in range(nk)
        ]
        sds = []
        for k in range(nk):
            gds[k].wait()
            sds.append(
                pltpu.async_copy(rows[k], acc.at[didx.at[k, 0]], ssem,
                                 add=True))
        for d in sds:
            d.wait()

    def group_body(g, carry):
        da = pltpu.async_copy(src_hbm.at[s, pl.ds(g * G, G)], sidx, gsem)
        db = pltpu.async_copy(dst_hbm.at[s, pl.ds(g * G, G)], didx, gsem)
        da.wait()
        db.wait()

        @pl.when(c == 0)
        def _():
            run(h0_hbm, G)

        @pl.when(c == 1)
        def _():
            run(h1_hbm, G)

        return carry

    lax.fori_loop(0, NGROUP, group_body, 0)

    if GTAIL:
        da = pltpu.async_copy(src_hbm.at[s, pl.ds(NGROUP * G, GTAIL)],
                              sidx.at[pl.ds(0, GTAIL)], gsem)
        db = pltpu.async_copy(dst_hbm.at[s, pl.ds(NGROUP * G, GTAIL)],
                              didx.at[pl.ds(0, GTAIL)], gsem)
        da.wait()
        db.wait()

        @pl.when(c == 0)
        def _():
            run(h0_hbm, GTAIL)

        @pl.when(c == 1)
        def _():
            run(h1_hbm, GTAIL)

    plsc.subcore_barrier()

    def run_wb(out_hbm):
        wrs = []
        for t in range(NWB):
            sl = pl.ds(base_row + t * WB, WB)
            buf = wbuf_a if t % 2 == 0 else wbuf_b
            if t >= 2:
                wrs[t - 2].wait()
            pltpu.sync_copy(acc.at[sl], buf)
            wrs.append(pltpu.async_copy(buf, out_hbm.at[sl], ssem))
        wrs[NWB - 2].wait()
        wrs[NWB - 1].wait()

        @pl.when(s == NS - 1)
        def _():
            sl = pl.ds(TAIL_BASE, TAIL)
            pltpu.sync_copy(acc.at[sl], wbuf_a.at[pl.ds(0, TAIL)])
            pltpu.sync_copy(wbuf_a.at[pl.ds(0, TAIL)], out_hbm.at[sl])

    @pl.when(c == 0)
    def _():
        run_wb(out0_hbm)

    @pl.when(c == 1)
    def _():
        run_wb(out1_hbm)


def _dinv_from_parts(deg_ref):
    deg = deg_ref[...].sum(axis=(0, 1, 2)) + 1.0
    return lax.rsqrt(deg)


def _tc1_body(x_ref, w_ref, deg_ref, h0_ref, h1_ref):
    dinv = _dinv_from_parts(deg_ref)[:, None]
    h = jnp.dot(x_ref[...], w_ref[...], preferred_element_type=jnp.float32)
    hp = h * dinv
    h0_ref[...] = hp[:, :H]
    h1_ref[...] = hp[:, H:]


_tc1 = pl.pallas_call(
    _tc1_body,
    grid=(RB,),
    in_specs=[
        pl.BlockSpec((R, D), lambda r: (r, 0)),
        pl.BlockSpec((D, D), lambda r: (0, 0)),
        pl.BlockSpec((NC * NS, 1, 1, R), lambda r: (0, r, 0, 0)),
    ],
    out_specs=(
        pl.BlockSpec((R, H), lambda r: (r, 0)),
        pl.BlockSpec((R, H), lambda r: (r, 0)),
    ),
    out_shape=(
        jax.ShapeDtypeStruct((N, H), jnp.float32),
        jax.ShapeDtypeStruct((N, H), jnp.float32),
    ),
)


def _tc2_body(a0_ref, a1_ref, p0_ref, p1_ref, deg_ref, b1_ref, w2_ref,
              o0_ref, o1_ref):
    dinv = _dinv_from_parts(deg_ref)[:, None]
    b1 = b1_ref[...]
    r0 = jnp.maximum(dinv * (a0_ref[...] + p0_ref[...]) + b1[0][None, :], 0.0)
    r1 = jnp.maximum(dinv * (a1_ref[...] + p1_ref[...]) + b1[1][None, :], 0.0)
    r = jnp.concatenate([r0, r1], axis=1)
    h = jnp.dot(r, w2_ref[...], preferred_element_type=jnp.float32)
    hp = h * dinv
    o0_ref[...] = hp[:, :H]
    o1_ref[...] = hp[:, H:]


_tc2 = pl.pallas_call(
    _tc2_body,
    grid=(RB,),
    in_specs=[
        pl.BlockSpec((R, H), lambda r: (r, 0)),
        pl.BlockSpec((R, H), lambda r: (r, 0)),
        pl.BlockSpec((R, H), lambda r: (r, 0)),
        pl.BlockSpec((R, H), lambda r: (r, 0)),
        pl.BlockSpec((NC * NS, 1, 1, R), lambda r: (0, r, 0, 0)),
        pl.BlockSpec((2, H), lambda r: (0, 0)),
        pl.BlockSpec((D, D), lambda r: (0, 0)),
    ],
    out_specs=(
        pl.BlockSpec((R, H), lambda r: (r, 0)),
        pl.BlockSpec((R, H), lambda r: (r, 0)),
    ),
    out_shape=(
        jax.ShapeDtypeStruct((N, H), jnp.float32),
        jax.ShapeDtypeStruct((N, H), jnp.float32),
    ),
)


def _tc3_body(a0_ref, a1_ref, p0_ref, p1_ref, deg_ref, b2_ref, out_ref):
    dinv = _dinv_from_parts(deg_ref)[:, None]
    b2 = b2_ref[...]
    o0 = dinv * (a0_ref[...] + p0_ref[...]) + b2[0][None, :]
    o1 = dinv * (a1_ref[...] + p1_ref[...]) + b2[1][None, :]
    out_ref[...] = jnp.concatenate([o0, o1], axis=1)


_tc3 = pl.pallas_call(
    _tc3_body,
    grid=(RB,),
    in_specs=[
        pl.BlockSpec((R, H), lambda r: (r, 0)),
        pl.BlockSpec((R, H), lambda r: (r, 0)),
        pl.BlockSpec((R, H), lambda r: (r, 0)),
        pl.BlockSpec((R, H), lambda r: (r, 0)),
        pl.BlockSpec((NC * NS, 1, 1, R), lambda r: (0, r, 0, 0)),
        pl.BlockSpec((2, H), lambda r: (0, 0)),
    ],
    out_specs=pl.BlockSpec((R, D), lambda r: (r, 0)),
    out_shape=jax.ShapeDtypeStruct((N, D), jnp.float32),
)


def kernel(x, edge_index, W1, b1, W2, b2):
    src = edge_index[0].astype(jnp.int32)
    dst = edge_index[1].astype(jnp.int32)

    deg_parts = _deg_kernel(dst.reshape(NC * NS, 1, E_TILE_DEG))
    deg_parts = deg_parts.reshape(NC * NS, RB, 1, R)

    src_r = src.reshape(NS, NCHUNK, 1, CHUNK)
    dst_r = dst.reshape(NS, NCHUNK, 1, CHUNK)

    h0, h1 = _tc1(x, W1, deg_parts)
    a0, a1 = _agg_kernel(h0, h1, src_r, dst_r)
    g0, g1 = _tc2(a0, a1, h0, h1, deg_parts, b1.reshape(2, H), W2)
    c0, c1 = _agg_kernel(g0, g1, src_r, dst_r)
    return _tc3(c0, c1, g0, g1, deg_parts, b2.reshape(2, H))
